# Initial kernel scaffold; baseline (speedup 1.0000x reference)
import jax
import jax.numpy as jnp
from jax.experimental import pallas as pl


def kernel(all_emb, W_lawm, static_weights, alpha, edge_index, edge_weight):
    raise NotImplementedError("write your pallas kernel here")


# speedup vs baseline: 4.1564x; 4.1564x over previous
import functools

import jax
import jax.numpy as jnp
from jax import lax
from jax.experimental import pallas as pl
from jax.experimental.pallas import tpu as pltpu
from jax.experimental.pallas import tpu_sc as plsc

N = 10000
E = 320000
D = 128
L = 3

NC = 2
NS = 16
NW = NC * NS
EP = E // NW
C = 80
NCHUNK = EP // C
RPT = 624
RTAIL = N - NS * RPT

BLK = 2000
GRID = N // BLK


def _make_prop():
    mesh = plsc.VectorSubcoreMesh(core_axis_name="c", subcore_axis_name="s")

    @functools.partial(
        pl.kernel,
        mesh=mesh,
        out_type=jax.ShapeDtypeStruct((NC, N, D), jnp.float32),
        scratch_types=[
            pltpu.VMEM((C,), jnp.int32),
            pltpu.VMEM((C,), jnp.int32),
            pltpu.VMEM((C,), jnp.float32),
            pltpu.VMEM((C, D), jnp.float32),
            pltpu.VMEM_SHARED((N, D), jnp.float32),
            pltpu.SemaphoreType.DMA,
        ],
    )
    def prop(h_hbm, src_hbm, dst_hbm, w_hbm, zero_hbm, out_hbm,
             src_v, dst_v, w_v, rows_v, acc_sh, sem):
        c = lax.axis_index("c")
        s = lax.axis_index("s")
        wid = c * NS + s
        pltpu.sync_copy(zero_hbm.at[pl.ds(s * RPT, RPT)],
                        acc_sh.at[pl.ds(s * RPT, RPT)])

        @pl.when(s == NS - 1)
        def _():
            pltpu.sync_copy(zero_hbm.at[pl.ds(NS * RPT, RTAIL)],
                            acc_sh.at[pl.ds(NS * RPT, RTAIL)])

        plsc.subcore_barrier()
        base = wid * EP

        def chunk(i, carry):
            off = base + i * C
            pltpu.sync_copy(src_hbm.at[pl.ds(off, C)], src_v)
            pltpu.sync_copy(dst_hbm.at[pl.ds(off, C)], dst_v)
            pltpu.sync_copy(w_hbm.at[pl.ds(off, C)], w_v)
            pltpu.async_copy(h_hbm.at[src_v], rows_v, sem).wait()

            def scale(g, carry2):
                wv = w_v[pl.ds(g * 16, 16)]
                for j in range(16):
                    wsc = wv[j]
                    e = g * 16 + j
                    for dd in range(D // 16):
                        sl = pl.ds(dd * 16, 16)
                        rows_v[e, sl] = rows_v[e, sl] * wsc
                return carry2

            lax.fori_loop(0, C // 16, scale, 0)
            pltpu.sync_copy(rows_v, acc_sh.at[dst_v], add=True)
            return carry

        lax.fori_loop(0, NCHUNK, chunk, 0)
        plsc.subcore_barrier()
        pltpu.sync_copy(acc_sh.at[pl.ds(s * RPT, RPT)],
                        out_hbm.at[c, pl.ds(s * RPT, RPT)])

        @pl.when(s == NS - 1)
        def _():
            pltpu.sync_copy(acc_sh.at[pl.ds(NS * RPT, RTAIL)],
                            out_hbm.at[c, pl.ds(NS * RPT, RTAIL)])

    return prop


def _combine(part):
    def body(p_ref, o_ref):
        o_ref[...] = p_ref[0] + p_ref[1]

    return pl.pallas_call(
        body,
        grid=(GRID,),
        in_specs=[pl.BlockSpec((NC, BLK, D), lambda i: (0, i, 0))],
        out_specs=pl.BlockSpec((BLK, D), lambda i: (i, 0)),
        out_shape=jax.ShapeDtypeStruct((N, D), jnp.float32),
    )(part)


def _final(x, p1, p2, part3, static_weights, W_lawm, alpha):
    def body(al_ref, x_ref, p1_ref, p2_ref, p3_ref, sw_ref, wl_ref,
             o_ref, m_scr):
        @pl.when(pl.program_id(0) == 0)
        def _():
            e0 = jnp.exp(al_ref[0])
            e1 = jnp.exp(al_ref[1])
            e2 = jnp.exp(al_ref[2])
            e3 = jnp.exp(al_ref[3])
            inv = 1.0 / (e0 + e1 + e2 + e3)
            wl = wl_ref[...]
            wc0 = jnp.dot(sw_ref[0], wl, preferred_element_type=jnp.float32)
            wc1 = jnp.dot(sw_ref[1], wl, preferred_element_type=jnp.float32)
            wc2 = jnp.dot(sw_ref[2], wl, preferred_element_type=jnp.float32)
            q2 = jnp.dot(wc0, wc1, preferred_element_type=jnp.float32)
            q3 = jnp.dot(q2, wc2, preferred_element_type=jnp.float32)
            rr = lax.broadcasted_iota(jnp.int32, (D, D), 0)
            cc = lax.broadcasted_iota(jnp.int32, (D, D), 1)
            eye = jnp.where(rr == cc, 1.0, 0.0).astype(jnp.float32)
            m_scr[0] = eye * (e0 * inv)
            m_scr[1] = wc0 * (e1 * inv)
            m_scr[2] = q2 * (e2 * inv)
            m_scr[3] = q3 * (e3 * inv)

        p3 = p3_ref[0] + p3_ref[1]
        o_ref[...] = (
            jnp.dot(x_ref[...], m_scr[0], preferred_element_type=jnp.float32)
            + jnp.dot(p1_ref[...], m_scr[1], preferred_element_type=jnp.float32)
            + jnp.dot(p2_ref[...], m_scr[2], preferred_element_type=jnp.float32)
            + jnp.dot(p3, m_scr[3], preferred_element_type=jnp.float32)
        )

    row = lambda i: (i, 0)
    fixed2 = lambda i: (0, 0)
    return pl.pallas_call(
        body,
        grid=(GRID,),
        in_specs=[
            pl.BlockSpec(memory_space=pltpu.SMEM),
            pl.BlockSpec((BLK, D), row),
            pl.BlockSpec((BLK, D), row),
            pl.BlockSpec((BLK, D), row),
            pl.BlockSpec((NC, BLK, D), lambda i: (0, i, 0)),
            pl.BlockSpec((L, D, D), lambda i: (0, 0, 0)),
            pl.BlockSpec((D, D), fixed2),
        ],
        out_specs=pl.BlockSpec((BLK, D), row),
        out_shape=jax.ShapeDtypeStruct((N, D), jnp.float32),
        scratch_shapes=[pltpu.VMEM((4, D, D), jnp.float32)],
    )(alpha, x, p1, p2, part3, static_weights, W_lawm)


def kernel(all_emb, W_lawm, static_weights, alpha, edge_index, edge_weight):
    src = edge_index[0]
    dst = edge_index[1]
    zero = jnp.zeros((N, D), jnp.float32)
    prop = _make_prop()
    part1 = prop(all_emb, src, dst, edge_weight, zero)
    p1 = _combine(part1)
    part2 = prop(p1, src, dst, edge_weight, zero)
    p2 = _combine(part2)
    part3 = prop(p2, src, dst, edge_weight, zero)
    return _final(all_emb, p1, p2, part3, static_weights, W_lawm, alpha)


# speedup vs baseline: 10.8905x; 2.6202x over previous
import functools

import jax
import jax.numpy as jnp
from jax import lax
from jax.experimental import pallas as pl
from jax.experimental.pallas import tpu as pltpu
from jax.experimental.pallas import tpu_sc as plsc

N = 10000
E = 320000
D = 128
L = 3

NC = 2
NS = 16
NW = NC * NS
EP = E // NW
C = 80
NCHUNK = EP // C
RPT = 624
RTAIL = N - NS * RPT

BLK = 2000
GRID = N // BLK


def _make_prop():
    mesh = plsc.VectorSubcoreMesh(core_axis_name="c", subcore_axis_name="s")

    @functools.partial(
        pl.kernel,
        mesh=mesh,
        out_type=jax.ShapeDtypeStruct((NC, N, D), jnp.float32),
        scratch_types=[
            pltpu.VMEM((EP,), jnp.int32),
            pltpu.VMEM((EP,), jnp.int32),
            pltpu.VMEM((EP,), jnp.float32),
            pltpu.VMEM((C,), jnp.int32),
            pltpu.VMEM((C, D), jnp.float32),
            pltpu.VMEM((C, D), jnp.float32),
            pltpu.VMEM_SHARED((N, D), jnp.float32),
            pltpu.SemaphoreType.DMA,
            pltpu.SemaphoreType.DMA,
        ],
    )
    def prop(h_hbm, src_hbm, dst_hbm, w_hbm, zero_hbm, out_hbm,
             src_v, dst_v, w_v, dst_c, rows0, rows1, acc_sh, sem0, sem1):
        rows = (rows0, rows1)
        sems = (sem0, sem1)
        c = lax.axis_index("c")
        s = lax.axis_index("s")
        wid = c * NS + s
        pltpu.sync_copy(zero_hbm.at[pl.ds(s * RPT, RPT)],
                        acc_sh.at[pl.ds(s * RPT, RPT)])

        @pl.when(s == NS - 1)
        def _():
            pltpu.sync_copy(zero_hbm.at[pl.ds(NS * RPT, RTAIL)],
                            acc_sh.at[pl.ds(NS * RPT, RTAIL)])

        plsc.subcore_barrier()
        base = wid * EP
        pltpu.sync_copy(src_hbm.at[pl.ds(base, EP)], src_v)
        pltpu.sync_copy(dst_hbm.at[pl.ds(base, EP)], dst_v)
        pltpu.sync_copy(w_hbm.at[pl.ds(base, EP)], w_v)

        def start_gather(i, k):
            pltpu.async_copy(h_hbm.at[src_v.at[pl.ds(i * C, C)]],
                             rows[k], sems[k])

        def wait_gather(k):
            pltpu.make_async_copy(h_hbm.at[src_v.at[pl.ds(0, C)]],
                                  rows[k], sems[k]).wait()

        def process(i, k):
            rv = rows[k]

            def scale(g, carry2):
                wv = w_v[pl.ds(i * C + g * 16, 16)]
                for j in range(16):
                    wsc = wv[j]
                    e = g * 16 + j
                    for dd in range(D // 16):
                        sl = pl.ds(dd * 16, 16)
                        rv[e, sl] = rv[e, sl] * wsc
                return carry2

            lax.fori_loop(0, C // 16, scale, 0)
            for g in range(C // 16):
                dst_c[pl.ds(g * 16, 16)] = dst_v[pl.ds(i * C + g * 16, 16)]
            pltpu.sync_copy(rv, acc_sh.at[dst_c], add=True)

        start_gather(0, 0)
        start_gather(1, 1)

        def pair(o, carry):
            for k in range(2):
                i = 2 * o + k
                wait_gather(k)
                process(i, k)

                @pl.when(i + 2 < NCHUNK)
                def _():
                    start_gather(i + 2, k)
            return carry

        lax.fori_loop(0, NCHUNK // 2, pair, 0)
        wait_gather(0)
        process(NCHUNK - 1, 0)
        plsc.subcore_barrier()
        pltpu.sync_copy(acc_sh.at[pl.ds(s * RPT, RPT)],
                        out_hbm.at[c, pl.ds(s * RPT, RPT)])

        @pl.when(s == NS - 1)
        def _():
            pltpu.sync_copy(acc_sh.at[pl.ds(NS * RPT, RTAIL)],
                            out_hbm.at[c, pl.ds(NS * RPT, RTAIL)])

    return prop


def _combine(part):
    def body(p_ref, o_ref):
        o_ref[...] = p_ref[0] + p_ref[1]

    return pl.pallas_call(
        body,
        grid=(GRID,),
        in_specs=[pl.BlockSpec((NC, BLK, D), lambda i: (0, i, 0))],
        out_specs=pl.BlockSpec((BLK, D), lambda i: (i, 0)),
        out_shape=jax.ShapeDtypeStruct((N, D), jnp.float32),
    )(part)


def _final(x, p1, p2, part3, static_weights, W_lawm, alpha):
    def body(al_ref, x_ref, p1_ref, p2_ref, p3_ref, sw_ref, wl_ref,
             o_ref, m_scr):
        @pl.when(pl.program_id(0) == 0)
        def _():
            e0 = jnp.exp(al_ref[0])
            e1 = jnp.exp(al_ref[1])
            e2 = jnp.exp(al_ref[2])
            e3 = jnp.exp(al_ref[3])
            inv = 1.0 / (e0 + e1 + e2 + e3)
            wl = wl_ref[...]
            wc0 = jnp.dot(sw_ref[0], wl, preferred_element_type=jnp.float32)
            wc1 = jnp.dot(sw_ref[1], wl, preferred_element_type=jnp.float32)
            wc2 = jnp.dot(sw_ref[2], wl, preferred_element_type=jnp.float32)
            q2 = jnp.dot(wc0, wc1, preferred_element_type=jnp.float32)
            q3 = jnp.dot(q2, wc2, preferred_element_type=jnp.float32)
            rr = lax.broadcasted_iota(jnp.int32, (D, D), 0)
            cc = lax.broadcasted_iota(jnp.int32, (D, D), 1)
            eye = jnp.where(rr == cc, 1.0, 0.0).astype(jnp.float32)
            m_scr[0] = eye * (e0 * inv)
            m_scr[1] = wc0 * (e1 * inv)
            m_scr[2] = q2 * (e2 * inv)
            m_scr[3] = q3 * (e3 * inv)

        p3 = p3_ref[0] + p3_ref[1]
        o_ref[...] = (
            jnp.dot(x_ref[...], m_scr[0], preferred_element_type=jnp.float32)
            + jnp.dot(p1_ref[...], m_scr[1], preferred_element_type=jnp.float32)
            + jnp.dot(p2_ref[...], m_scr[2], preferred_element_type=jnp.float32)
            + jnp.dot(p3, m_scr[3], preferred_element_type=jnp.float32)
        )

    row = lambda i: (i, 0)
    fixed2 = lambda i: (0, 0)
    return pl.pallas_call(
        body,
        grid=(GRID,),
        in_specs=[
            pl.BlockSpec(memory_space=pltpu.SMEM),
            pl.BlockSpec((BLK, D), row),
            pl.BlockSpec((BLK, D), row),
            pl.BlockSpec((BLK, D), row),
            pl.BlockSpec((NC, BLK, D), lambda i: (0, i, 0)),
            pl.BlockSpec((L, D, D), lambda i: (0, 0, 0)),
            pl.BlockSpec((D, D), fixed2),
        ],
        out_specs=pl.BlockSpec((BLK, D), row),
        out_shape=jax.ShapeDtypeStruct((N, D), jnp.float32),
        scratch_shapes=[pltpu.VMEM((4, D, D), jnp.float32)],
    )(alpha, x, p1, p2, part3, static_weights, W_lawm)


def kernel(all_emb, W_lawm, static_weights, alpha, edge_index, edge_weight):
    src = edge_index[0]
    dst = edge_index[1]
    zero = jnp.zeros((N, D), jnp.float32)
    prop = _make_prop()
    part1 = prop(all_emb, src, dst, edge_weight, zero)
    p1 = _combine(part1)
    part2 = prop(p1, src, dst, edge_weight, zero)
    p2 = _combine(part2)
    part3 = prop(p2, src, dst, edge_weight, zero)
    return _final(all_emb, p1, p2, part3, static_weights, W_lawm, alpha)
